# Initial kernel scaffold; baseline (speedup 1.0000x reference)
import jax
import jax.numpy as jnp
from jax.experimental import pallas as pl


def kernel(x, emb):
    raise NotImplementedError("write your pallas kernel here")


# speedup vs baseline: 3.0205x; 3.0205x over previous
import jax
import jax.numpy as jnp
from jax.experimental import pallas as pl


_BLOCK_ROWS = 1024


def _scale_copy_kernel(emb_ref, out_ref, *, scale):
    out_ref[...] = emb_ref[...] * scale


def kernel(x, emb):
    seq_len = x.shape[1]
    dim = emb.shape[1]
    scale = float(dim) ** -0.5
    table = emb[:seq_len]
    rows = table.shape[0]
    block_rows = min(_BLOCK_ROWS, rows)
    grid = (rows // block_rows,)
    import functools
    body = functools.partial(_scale_copy_kernel, scale=scale)
    return pl.pallas_call(
        body,
        grid=grid,
        in_specs=[pl.BlockSpec((block_rows, dim), lambda i: (i, 0))],
        out_specs=pl.BlockSpec((block_rows, dim), lambda i: (i, 0)),
        out_shape=jax.ShapeDtypeStruct((rows, dim), emb.dtype),
    )(table)
